# Initial kernel scaffold; baseline (speedup 1.0000x reference)
import jax
import jax.numpy as jnp
from jax.experimental import pallas as pl


def kernel(unary, binary, deltas, index1, index2):
    raise NotImplementedError("write your pallas kernel here")


# speedup vs baseline: 1.7822x; 1.7822x over previous
import functools

import jax
import jax.numpy as jnp
from jax import lax
from jax.experimental import pallas as pl
from jax.experimental.pallas import tpu as pltpu
from jax.experimental.pallas import tpu_sc as plsc

N_U = 128
N_B = 16
N_NODES = 10000
N_EDGES = 320000
NC = 2
NS = 16
NW = NC * NS
EDGES_PER_TILE = N_EDGES // NW
W = 80
N_WIN = EDGES_PER_TILE // W
N_NODES_PAD = 10240
ACC_ROWS_PER_TILE = N_NODES_PAD // NS
ZCHUNK = 128


def _sc_body(deltas, idx1, idx2, partial, b_out,
             ux_buf, uy_buf, b_buf, i1_buf, i2_buf, stage, acc):
    cid = lax.axis_index("c")
    sid = lax.axis_index("s")
    wid = sid * NC + cid

    def _zero_stage(j, _):
        for k in range(N_U // 16):
            stage[j, pl.ds(16 * k, 16)] = jnp.zeros((16,), jnp.float32)
        return _
    lax.fori_loop(0, ZCHUNK, _zero_stage, None)
    for j in range(ACC_ROWS_PER_TILE // ZCHUNK):
        pltpu.sync_copy(
            stage, acc.at[pl.ds(sid * ACC_ROWS_PER_TILE + j * ZCHUNK, ZCHUNK), :])
    plsc.subcore_barrier()

    def _window(w, _):
        base = wid * EDGES_PER_TILE + w * W
        pltpu.sync_copy(idx1.at[pl.ds(base, W)], i1_buf)
        pltpu.sync_copy(idx2.at[pl.ds(base, W)], i2_buf)
        pltpu.sync_copy(deltas.at[pl.ds(base, W), pl.ds(0, N_U)], ux_buf)
        pltpu.sync_copy(deltas.at[pl.ds(base, W), pl.ds(N_U, N_U)], uy_buf)
        pltpu.sync_copy(deltas.at[pl.ds(base, W), pl.ds(2 * N_U, N_B)], b_buf)
        pltpu.sync_copy(b_buf, b_out.at[pl.ds(base, W), :])
        pltpu.sync_copy(ux_buf, acc.at[i1_buf], add=True)
        pltpu.sync_copy(uy_buf, acc.at[i2_buf], add=True)
        return _
    lax.fori_loop(0, N_WIN, _window, None)
    plsc.subcore_barrier()

    row0 = sid * ACC_ROWS_PER_TILE
    for j in range(ACC_ROWS_PER_TILE // ZCHUNK):
        r = row0 + j * ZCHUNK
        pltpu.sync_copy(acc.at[pl.ds(r, ZCHUNK), :], stage)
        pltpu.sync_copy(stage, partial.at[cid, pl.ds(r, ZCHUNK), :])


_sc_call = functools.partial(
    pl.kernel,
    out_type=[
        jax.ShapeDtypeStruct((NC, N_NODES_PAD, N_U), jnp.float32),
        jax.ShapeDtypeStruct((N_EDGES, N_B), jnp.float32),
    ],
    mesh=plsc.VectorSubcoreMesh(core_axis_name="c", subcore_axis_name="s"),
    scratch_types=[
        pltpu.VMEM((W, N_U), jnp.float32),
        pltpu.VMEM((W, N_U), jnp.float32),
        pltpu.VMEM((W, N_B), jnp.float32),
        pltpu.VMEM((W,), jnp.int32),
        pltpu.VMEM((W,), jnp.int32),
        pltpu.VMEM((ZCHUNK, N_U), jnp.float32),
        pltpu.VMEM_SHARED((N_NODES_PAD, N_U), jnp.float32),
    ],
)(_sc_body)


def _add_body(p_ref, o_ref):
    o_ref[...] = p_ref[0] + p_ref[1]


def _combine_partials(partial):
    blk = 1000
    return pl.pallas_call(
        _add_body,
        out_shape=jax.ShapeDtypeStruct((N_NODES, N_U), jnp.float32),
        grid=(N_NODES // blk,),
        in_specs=[pl.BlockSpec((NC, blk, N_U), lambda i: (0, i, 0))],
        out_specs=pl.BlockSpec((blk, N_U), lambda i: (i, 0)),
    )(partial)


def kernel(unary, binary, deltas, index1, index2):
    idx1 = index1.astype(jnp.int32)
    idx2 = index2.astype(jnp.int32)
    partial, b = _sc_call(deltas, idx1, idx2)
    return (_combine_partials(partial), b)


# speedup vs baseline: 2.7734x; 1.5562x over previous
import functools

import jax
import jax.numpy as jnp
from jax import lax
from jax.experimental import pallas as pl
from jax.experimental.pallas import tpu as pltpu
from jax.experimental.pallas import tpu_sc as plsc

N_U = 128
N_B = 16
N_NODES = 10000
N_EDGES = 320000
NC = 2
NS = 16
NW = NC * NS
EDGES_PER_TILE = N_EDGES // NW
W = 40
N_WIN = EDGES_PER_TILE // W
N_NODES_PAD = 10240
ACC_ROWS_PER_TILE = N_NODES_PAD // NS
ZCHUNK = 128


def _sc_body(deltas, idx1, idx2, partial, b_out,
             ux0, ux1, uy0, uy1, b0, b1, i10, i11, i20, i21, stage, acc,
             ld_sem0, ld_sem1, bw_sem0, bw_sem1, sc_sem):
    cid = lax.axis_index("c")
    sid = lax.axis_index("s")
    wid = sid * NC + cid

    ux = (ux0, ux1)
    uy = (uy0, uy1)
    bb = (b0, b1)
    i1 = (i10, i11)
    i2 = (i20, i21)
    ld_sem = (ld_sem0, ld_sem1)
    bw_sem = (bw_sem0, bw_sem1)

    def _zero_stage(j, _):
        for k in range(N_U // 16):
            stage[j, pl.ds(16 * k, 16)] = jnp.zeros((16,), jnp.float32)
        return _
    lax.fori_loop(0, ZCHUNK, _zero_stage, None)
    for j in range(ACC_ROWS_PER_TILE // ZCHUNK):
        pltpu.sync_copy(
            stage, acc.at[pl.ds(sid * ACC_ROWS_PER_TILE + j * ZCHUNK, ZCHUNK), :])
    plsc.subcore_barrier()

    def _base(w):
        return wid * EDGES_PER_TILE + w * W

    def _issue(w, p):
        base = _base(w)
        pltpu.async_copy(idx1.at[pl.ds(base, W)], i1[p], ld_sem[p])
        pltpu.async_copy(idx2.at[pl.ds(base, W)], i2[p], ld_sem[p])
        pltpu.async_copy(deltas.at[pl.ds(base, W), pl.ds(0, N_U)], ux[p], ld_sem[p])
        pltpu.async_copy(deltas.at[pl.ds(base, W), pl.ds(N_U, N_U)], uy[p], ld_sem[p])
        pltpu.async_copy(deltas.at[pl.ds(base, W), pl.ds(2 * N_U, N_B)], bb[p], ld_sem[p])

    def _wait_loads(w, p):
        base = _base(w)
        pltpu.make_async_copy(idx1.at[pl.ds(base, W)], i1[p], ld_sem[p]).wait()
        pltpu.make_async_copy(idx2.at[pl.ds(base, W)], i2[p], ld_sem[p]).wait()
        pltpu.make_async_copy(deltas.at[pl.ds(base, W), pl.ds(0, N_U)], ux[p], ld_sem[p]).wait()
        pltpu.make_async_copy(deltas.at[pl.ds(base, W), pl.ds(N_U, N_U)], uy[p], ld_sem[p]).wait()
        pltpu.make_async_copy(deltas.at[pl.ds(base, W), pl.ds(2 * N_U, N_B)], bb[p], ld_sem[p]).wait()

    def _wait_bwrite(w, p):
        base = _base(w)
        pltpu.make_async_copy(bb[p], b_out.at[pl.ds(base, W), :], bw_sem[p]).wait()

    def _consume(w, p):
        _wait_loads(w, p)
        base = _base(w)
        pltpu.async_copy(bb[p], b_out.at[pl.ds(base, W), :], bw_sem[p])
        a1 = pltpu.async_copy(ux[p], acc.at[i1[p]], sc_sem, add=True)
        a2 = pltpu.async_copy(uy[p], acc.at[i2[p]], sc_sem, add=True)
        a1.wait()
        a2.wait()

    _issue(0, 0)

    def _body(i, _):
        w0 = 2 * i
        w1 = w0 + 1
        w2 = w0 + 2

        @pl.when(i > 0)
        def _():
            _wait_bwrite(w1 - 2, 1)

        @pl.when(w1 < N_WIN)
        def _():
            _issue(w1, 1)
        _consume(w0, 0)
        _wait_bwrite(w0, 0)

        @pl.when(w2 < N_WIN)
        def _():
            _issue(w2, 0)

        @pl.when(w1 < N_WIN)
        def _():
            _consume(w1, 1)
        return _

    lax.fori_loop(0, (N_WIN + 1) // 2, _body, None)
    if N_WIN % 2 == 0:
        _wait_bwrite(N_WIN - 1, 1)
    plsc.subcore_barrier()

    row0 = sid * ACC_ROWS_PER_TILE
    for j in range(ACC_ROWS_PER_TILE // ZCHUNK):
        r = row0 + j * ZCHUNK
        pltpu.sync_copy(acc.at[pl.ds(r, ZCHUNK), :], stage)
        pltpu.sync_copy(stage, partial.at[cid, pl.ds(r, ZCHUNK), :])


_sc_call = functools.partial(
    pl.kernel,
    out_type=[
        jax.ShapeDtypeStruct((NC, N_NODES_PAD, N_U), jnp.float32),
        jax.ShapeDtypeStruct((N_EDGES, N_B), jnp.float32),
    ],
    mesh=plsc.VectorSubcoreMesh(core_axis_name="c", subcore_axis_name="s"),
    scratch_types=[
        pltpu.VMEM((W, N_U), jnp.float32),
        pltpu.VMEM((W, N_U), jnp.float32),
        pltpu.VMEM((W, N_U), jnp.float32),
        pltpu.VMEM((W, N_U), jnp.float32),
        pltpu.VMEM((W, N_B), jnp.float32),
        pltpu.VMEM((W, N_B), jnp.float32),
        pltpu.VMEM((W,), jnp.int32),
        pltpu.VMEM((W,), jnp.int32),
        pltpu.VMEM((W,), jnp.int32),
        pltpu.VMEM((W,), jnp.int32),
        pltpu.VMEM((ZCHUNK, N_U), jnp.float32),
        pltpu.VMEM_SHARED((N_NODES_PAD, N_U), jnp.float32),
        pltpu.SemaphoreType.DMA,
        pltpu.SemaphoreType.DMA,
        pltpu.SemaphoreType.DMA,
        pltpu.SemaphoreType.DMA,
        pltpu.SemaphoreType.DMA,
    ],
)(_sc_body)


def _add_body(p_ref, o_ref):
    o_ref[...] = p_ref[0] + p_ref[1]


def _combine_partials(partial):
    blk = 1000
    return pl.pallas_call(
        _add_body,
        out_shape=jax.ShapeDtypeStruct((N_NODES, N_U), jnp.float32),
        grid=(N_NODES // blk,),
        in_specs=[pl.BlockSpec((NC, blk, N_U), lambda i: (0, i, 0))],
        out_specs=pl.BlockSpec((blk, N_U), lambda i: (i, 0)),
    )(partial)


def kernel(unary, binary, deltas, index1, index2):
    idx1 = index1.astype(jnp.int32)
    idx2 = index2.astype(jnp.int32)
    partial, b = _sc_call(deltas, idx1, idx2)
    return (_combine_partials(partial), b)


# speedup vs baseline: 3.1477x; 1.1350x over previous
import functools

import jax
import jax.numpy as jnp
from jax import lax
from jax.experimental import pallas as pl
from jax.experimental.pallas import tpu as pltpu
from jax.experimental.pallas import tpu_sc as plsc

N_U = 128
N_B = 16
N_NODES = 10000
N_EDGES = 320000
NC = 2
NS = 16
NW = NC * NS
EDGES_PER_TILE = N_EDGES // NW
W = 40
N_WIN = EDGES_PER_TILE // W
NBUF = 4
N_NODES_PAD = 10240
ACC_ROWS_PER_TILE = N_NODES_PAD // NS


def _sc_body(deltas, idx1, idx2, partial,
             ux0, ux1, ux2, ux3, uy0, uy1, uy2, uy3,
             i10, i11, i12, i13, i20, i21, i22, i23, acc,
             ld_sem0, ld_sem1, ld_sem2, ld_sem3, sc_sem):
    cid = lax.axis_index("c")
    sid = lax.axis_index("s")
    wid = sid * NC + cid

    ux = (ux0, ux1, ux2, ux3)
    uy = (uy0, uy1, uy2, uy3)
    i1 = (i10, i11, i12, i13)
    i2 = (i20, i21, i22, i23)
    ld_sem = (ld_sem0, ld_sem1, ld_sem2, ld_sem3)

    def _zero_stage(j, _):
        for k in range(N_U // 16):
            ux0[j, pl.ds(16 * k, 16)] = jnp.zeros((16,), jnp.float32)
        return _
    lax.fori_loop(0, W, _zero_stage, None)
    for j in range(ACC_ROWS_PER_TILE // W):
        pltpu.sync_copy(
            ux0, acc.at[pl.ds(sid * ACC_ROWS_PER_TILE + j * W, W), :])
    plsc.subcore_barrier()

    def _base(w):
        return wid * EDGES_PER_TILE + w * W

    def _issue(w, p):
        base = _base(w)
        pltpu.async_copy(idx1.at[pl.ds(base, W)], i1[p], ld_sem[p])
        pltpu.async_copy(idx2.at[pl.ds(base, W)], i2[p], ld_sem[p])
        pltpu.async_copy(deltas.at[pl.ds(base, W), pl.ds(0, N_U)], ux[p], ld_sem[p])
        pltpu.async_copy(deltas.at[pl.ds(base, W), pl.ds(N_U, N_U)], uy[p], ld_sem[p])

    def _consume(w, p):
        base = _base(w)
        pltpu.make_async_copy(idx1.at[pl.ds(base, W)], i1[p], ld_sem[p]).wait()
        pltpu.make_async_copy(idx2.at[pl.ds(base, W)], i2[p], ld_sem[p]).wait()
        pltpu.make_async_copy(deltas.at[pl.ds(base, W), pl.ds(0, N_U)], ux[p], ld_sem[p]).wait()
        pltpu.make_async_copy(deltas.at[pl.ds(base, W), pl.ds(N_U, N_U)], uy[p], ld_sem[p]).wait()
        a1 = pltpu.async_copy(ux[p], acc.at[i1[p]], sc_sem, add=True)
        a2 = pltpu.async_copy(uy[p], acc.at[i2[p]], sc_sem, add=True)
        a1.wait()
        a2.wait()

    for w in range(NBUF - 1):
        _issue(w, w)

    def _body(i, _):
        for k in range(NBUF):
            w = NBUF * i + k
            _consume(w, k)
            wn = w + NBUF - 1

            @pl.when(wn < N_WIN)
            def _():
                _issue(wn, (k + NBUF - 1) % NBUF)
        return _

    lax.fori_loop(0, N_WIN // NBUF, _body, None)
    for w in range(N_WIN - N_WIN % NBUF, N_WIN):
        _consume(w, w % NBUF)
    plsc.subcore_barrier()

    row0 = sid * ACC_ROWS_PER_TILE
    for j in range(ACC_ROWS_PER_TILE // W):
        r = row0 + j * W
        pltpu.sync_copy(acc.at[pl.ds(r, W), :], ux0)
        pltpu.sync_copy(ux0, partial.at[cid, pl.ds(r, W), :])


def _make_sc_call():
    scratch = (
        [pltpu.VMEM((W, N_U), jnp.float32) for _ in range(2 * NBUF)]
        + [pltpu.VMEM((W,), jnp.int32) for _ in range(2 * NBUF)]
        + [pltpu.VMEM_SHARED((N_NODES_PAD, N_U), jnp.float32)]
        + [pltpu.SemaphoreType.DMA for _ in range(NBUF + 1)]
    )
    return functools.partial(
        pl.kernel,
        out_type=[
            jax.ShapeDtypeStruct((NC, N_NODES_PAD, N_U), jnp.float32),
        ],
        mesh=plsc.VectorSubcoreMesh(core_axis_name="c", subcore_axis_name="s"),
        scratch_types=scratch,
    )(_sc_body)


_sc_call = _make_sc_call()


def _add_body(p_ref, o_ref):
    o_ref[...] = p_ref[0] + p_ref[1]


def _combine_partials(partial):
    blk = 1000
    return pl.pallas_call(
        _add_body,
        out_shape=jax.ShapeDtypeStruct((N_NODES, N_U), jnp.float32),
        grid=(N_NODES // blk,),
        in_specs=[pl.BlockSpec((NC, blk, N_U), lambda i: (0, i, 0))],
        out_specs=pl.BlockSpec((blk, N_U), lambda i: (i, 0)),
    )(partial)


def _b_body(d_ref, o_ref):
    o_ref[...] = d_ref[:, :N_B]


def _copy_b(deltas):
    blk = 8000
    return pl.pallas_call(
        _b_body,
        out_shape=jax.ShapeDtypeStruct((N_EDGES, N_B), jnp.float32),
        grid=(N_EDGES // blk,),
        in_specs=[pl.BlockSpec((blk, 128), lambda i: (i, 2))],
        out_specs=pl.BlockSpec((blk, N_B), lambda i: (i, 0)),
    )(deltas)


def kernel(unary, binary, deltas, index1, index2):
    idx1 = index1.astype(jnp.int32)
    idx2 = index2.astype(jnp.int32)
    outs = _sc_call(deltas, idx1, idx2)
    partial = outs[0] if isinstance(outs, (tuple, list)) else outs
    return (_combine_partials(partial), _copy_b(deltas))


# speedup vs baseline: 5.7197x; 1.8171x over previous
import functools

import jax
import jax.numpy as jnp
from jax import lax
from jax.experimental import pallas as pl
from jax.experimental.pallas import tpu as pltpu
from jax.experimental.pallas import tpu_sc as plsc

N_U = 128
N_B = 16
N_NODES = 10000
N_EDGES = 320000
NC = 2
NS = 16
FPT = 8
E = 1280
N_WIN = N_EDGES // E
N_NODES_PAD = 10240
BINS_STRIDE = N_NODES_PAD + 17


def _sc_body(deltasT, idx_cat, partialT,
             vb0, vb1, vb2, ib0, ib1, ib2, bins,
             ld_sem0, ld_sem1, ld_sem2):
    cid = lax.axis_index("c")
    sid = lax.axis_index("s")
    fbase = cid * N_U + FPT * sid

    vb = (vb0, vb1, vb2)
    ib = (ib0, ib1, ib2)
    ld_sem = (ld_sem0, ld_sem1, ld_sem2)
    f_vecs = [jnp.full((16,), f, jnp.int32) for f in range(FPT)]
    idx0 = cid * N_EDGES

    def _issue(w, p):
        e0 = w * E
        pltpu.async_copy(idx_cat.at[pl.ds(idx0 + e0, E)], ib[p], ld_sem[p])
        pltpu.async_copy(
            deltasT.at[pl.ds(fbase, FPT), pl.ds(e0, E)], vb[p], ld_sem[p])

    def _consume(w, p):
        e0 = w * E
        pltpu.make_async_copy(
            idx_cat.at[pl.ds(idx0 + e0, E)], ib[p], ld_sem[p]).wait()
        pltpu.make_async_copy(
            deltasT.at[pl.ds(fbase, FPT), pl.ds(e0, E)], vb[p], ld_sem[p]).wait()

        UNROLL = 4

        def _blk(g, _):
            idxvs = [
                ib[p][pl.ds(16 * UNROLL * g + 16 * u, 16)]
                for u in range(UNROLL)
            ]
            valss = [
                [vb[p][f, pl.ds(16 * UNROLL * g + 16 * u, 16)]
                 for f in range(FPT)]
                for u in range(UNROLL)
            ]
            for u in range(UNROLL):
                for f in range(FPT):
                    plsc.addupdate_scatter(
                        bins, [f_vecs[f], idxvs[u]], valss[u][f])
            return _
        lax.fori_loop(0, E // (16 * UNROLL), _blk, None)

    _issue(0, 0)
    _issue(1, 1)

    def _zero(j, _):
        for f in range(FPT):
            for k in range(8):
                bins[f, pl.ds(128 * j + 16 * k, 16)] = jnp.zeros(
                    (16,), jnp.float32)
        return _
    lax.fori_loop(0, N_NODES_PAD // 128, _zero, None)

    def _body(i, _):
        for k in range(3):
            w = 3 * i + k
            _consume(w, k)
            wn = w + 2

            @pl.when(wn < N_WIN)
            def _():
                _issue(wn, (k + 2) % 3)
        return _

    lax.fori_loop(0, N_WIN // 3, _body, None)
    for w in range(N_WIN - N_WIN % 3, N_WIN):
        _consume(w, w % 3)

    pltpu.sync_copy(bins, partialT.at[cid, pl.ds(FPT * sid, FPT), :])


_sc_call = functools.partial(
    pl.kernel,
    out_type=[
        jax.ShapeDtypeStruct((NC, N_U, BINS_STRIDE), jnp.float32),
    ],
    mesh=plsc.VectorSubcoreMesh(core_axis_name="c", subcore_axis_name="s"),
    compiler_params=pltpu.CompilerParams(needs_layout_passes=False),
    scratch_types=[
        pltpu.VMEM((FPT, E), jnp.float32),
        pltpu.VMEM((FPT, E), jnp.float32),
        pltpu.VMEM((FPT, E), jnp.float32),
        pltpu.VMEM((E,), jnp.int32),
        pltpu.VMEM((E,), jnp.int32),
        pltpu.VMEM((E,), jnp.int32),
        pltpu.VMEM((FPT, BINS_STRIDE), jnp.float32),
        pltpu.SemaphoreType.DMA,
        pltpu.SemaphoreType.DMA,
        pltpu.SemaphoreType.DMA,
    ],
)(_sc_body)


def _comb_body(p_ref, o_ref):
    o_ref[...] = jnp.transpose(p_ref[0] + p_ref[1])


def _combine_partials(partialT):
    blk = 1280
    return pl.pallas_call(
        _comb_body,
        out_shape=jax.ShapeDtypeStruct((N_NODES, N_U), jnp.float32),
        grid=(N_NODES_PAD // blk,),
        in_specs=[pl.BlockSpec((NC, N_U, blk), lambda i: (0, 0, i))],
        out_specs=pl.BlockSpec((blk, N_U), lambda i: (i, 0)),
    )(partialT)


def _b_body(d_ref, o_ref):
    o_ref[...] = d_ref[...]


def _copy_b(deltasT):
    blk = 32000
    bT = pl.pallas_call(
        _b_body,
        out_shape=jax.ShapeDtypeStruct((N_B, N_EDGES), jnp.float32),
        grid=(N_EDGES // blk,),
        in_specs=[pl.BlockSpec((N_B, blk), lambda i: (2 * N_U // N_B, i))],
        out_specs=pl.BlockSpec((N_B, blk), lambda i: (0, i)),
    )(deltasT)
    return jnp.transpose(bT)


def kernel(unary, binary, deltas, index1, index2):
    deltasT = jnp.transpose(deltas)
    idx_cat = jnp.concatenate(
        [index1.astype(jnp.int32), index2.astype(jnp.int32)])
    outs = _sc_call(deltasT, idx_cat)
    partialT = outs[0] if isinstance(outs, (tuple, list)) else outs
    return (_combine_partials(partialT), _copy_b(deltasT))
